# baseline (speedup 1.0000x reference)
import functools

import jax
import jax.numpy as jnp
from jax import lax
from jax.experimental import pallas as pl
from jax.experimental.pallas import tpu as pltpu
from jax.experimental.pallas import tpu_sc as plsc

N = 10000
E = 4096
NNZ = 320000
D = 128

CHUNK = 128
TPW = 80
R = 32 * TPW

EPT = E // 16
CROWS = E // D
VH = 5120
NSUB = 2
NP = NSUB * VH
ACC = 4112
ACCB = 5248
VPT = VH // 16

F32 = jnp.float32
I32 = jnp.int32

_MESH = dict(core_axis_name="c", subcore_axis_name="s")


_B1 = 400
_G1 = N // _B1


def _kmm_body(x_ref, w_ref, xw_ref):
    xw_ref[...] = jnp.dot(x_ref[...], w_ref[...], preferred_element_type=F32)


def _kernel_mm(X, W):
    return pl.pallas_call(
        _kmm_body,
        grid=(_G1,),
        in_specs=[
            pl.BlockSpec((_B1, D), lambda i: (i, 0)),
            pl.BlockSpec((D, D), lambda i: (0, 0)),
        ],
        out_specs=pl.BlockSpec((_B1, D), lambda i: (i, 0)),
        out_shape=jax.ShapeDtypeStruct((N, D), F32),
    )(X, W)


def _kdeg_body(h_ref, degv_ref, dege_ref):
    i = pl.program_id(0)
    h = h_ref[...]
    rv = jnp.sum(h, axis=1, keepdims=True)
    r = lax.rsqrt(rv)
    degv_ref[...] = jnp.where(jnp.isinf(r), 1.0, r)

    ce = jnp.sum(h, axis=0, keepdims=True)

    @pl.when(i == 0)
    def _():
        dege_ref[...] = jnp.zeros_like(dege_ref)

    dege_ref[...] += ce

    @pl.when(i == _G1 - 1)
    def _():
        dege_ref[...] = lax.rsqrt(dege_ref[...])


def _kernel_deg(H):
    return pl.pallas_call(
        _kdeg_body,
        grid=(_G1,),
        in_specs=[pl.BlockSpec((_B1, E), lambda i: (i, 0))],
        out_specs=[
            pl.BlockSpec((_B1, 1), lambda i: (i, 0)),
            pl.BlockSpec((1, E), lambda i: (0, 0)),
        ],
        out_shape=[
            jax.ShapeDtypeStruct((N, 1), F32),
            jax.ShapeDtypeStruct((1, E), F32),
        ],
    )(H)


def _sc_phase_a(pidx, table):
    mesh = plsc.VectorSubcoreMesh(**_MESH)

    @functools.partial(
        pl.kernel,
        out_type=(
            jax.ShapeDtypeStruct((2, E, D), F32),
            jax.ShapeDtypeStruct((2, CROWS, D), F32),
        ),
        mesh=mesh,
        compiler_params=pltpu.CompilerParams(needs_layout_passes=False),
        scratch_types=[
            pltpu.VMEM((TPW, CHUNK), I32),
            pltpu.VMEM((8, CHUNK), I32),
            pltpu.VMEM((8, CHUNK), I32),
            pltpu.VMEM((2, CHUNK, D), F32),
            pltpu.VMEM((EPT, D), F32),
            pltpu.VMEM((CROWS, D), F32),
            pltpu.VMEM_SHARED((ACC, D), F32),
            pltpu.SemaphoreType.DMA,
            pltpu.SemaphoreType.DMA,
        ],
    )
    def ka(pidx_hbm, table_hbm, sums_hbm, cnt_hbm,
           pix, idxg, idxs, gbuf, stage, hist, acc, gsem, ssem):
        c = lax.axis_index("c")
        s = lax.axis_index("s")
        base = (c * 16 + s) * TPW
        pltpu.sync_copy(pidx_hbm.at[pl.ds(base, TPW)], pix)

        def zstage(i, carry):
            for j in range(D // 16):
                stage[i, pl.ds(j * 16, 16)] = jnp.zeros((16,), F32)
            return carry

        lax.fori_loop(0, EPT, zstage, 0)

        @pl.when(s < 15)
        def _():
            pltpu.sync_copy(stage, acc.at[pl.ds(s * 264, 256)])
            pltpu.sync_copy(stage.at[pl.ds(0, 8)],
                            acc.at[pl.ds(s * 264 + 256, 8)])

        @pl.when(s == 15)
        def _():
            pltpu.sync_copy(stage.at[pl.ds(0, 152)], acc.at[pl.ds(3960, 152)])

        def zhist(i, carry):
            for j in range(D // 16):
                hist[i, pl.ds(j * 16, 16)] = jnp.zeros((16,), F32)
            return carry

        lax.fori_loop(0, CROWS, zhist, 0)
        plsc.subcore_barrier()

        def prep(t):
            w = t & 7
            for j in range(CHUNK // 16):
                pk = pix[t, pl.ds(j * 16, 16)]
                idxs[w, pl.ds(j * 16, 16)] = pk & 8191
                idxg[w, pl.ds(j * 16, 16)] = jnp.minimum(pk >> 13, N - 1)

        def fire_g(t):
            pltpu.async_copy(table_hbm.at[idxg.at[t & 7]], gbuf.at[t & 1],
                             gsem)

        def wait_g(t):
            pltpu.make_async_copy(
                table_hbm.at[idxg.at[t & 7]], gbuf.at[t & 1], gsem).wait()

        def fire_s(t):
            pltpu.async_copy(gbuf.at[t & 1], acc.at[idxs.at[t & 7]], ssem,
                             add=True)

        def wait_s(t):
            pltpu.make_async_copy(
                gbuf.at[t & 1], acc.at[idxs.at[t & 7]], ssem).wait()

        prep(0)
        fire_g(0)

        def body(t, carry):
            wait_g(t)
            fire_s(t)

            @pl.when(t >= 1)
            def _():
                wait_s(t - 1)

            @pl.when(t + 1 < TPW)
            def _():
                prep(t + 1)
                fire_g(t + 1)

            for j in range(CHUNK // 16):
                idx16 = pix[t, pl.ds(j * 16, 16)] & 8191
                plsc.addupdate_scatter(
                    hist, [idx16 >> 7, idx16 & 127], jnp.ones((16,), F32),
                    mask=idx16 < E)
            return carry

        lax.fori_loop(0, TPW, body, 0)
        wait_s(TPW - 1)
        plsc.subcore_barrier()

        pltpu.sync_copy(hist, sums_hbm.at[c, pl.ds(s * CROWS, CROWS)])
        plsc.subcore_barrier()

        @pl.when(s < 4)
        def _():
            for r in range(8):
                for j in range(D // 16):
                    hist[r, pl.ds(j * 16, 16)] = jnp.zeros((16,), F32)
            for src in range(16):
                pltpu.sync_copy(
                    sums_hbm.at[c, pl.ds(src * CROWS + 8 * s, 8)],
                    hist.at[pl.ds(8, 8)])
                for r in range(8):
                    for j in range(D // 16):
                        hist[r, pl.ds(j * 16, 16)] = (
                            hist[r, pl.ds(j * 16, 16)]
                            + hist[8 + r, pl.ds(j * 16, 16)])
            pltpu.sync_copy(hist.at[pl.ds(0, 8)],
                            cnt_hbm.at[c, pl.ds(8 * s, 8)])

        plsc.subcore_barrier()

        pltpu.sync_copy(acc.at[pl.ds(s * EPT, EPT)], stage)
        pltpu.sync_copy(stage, sums_hbm.at[c, pl.ds(s * EPT, EPT)])

    return ka(pidx, table)


def _sc_phase_b(pidx, xep):
    mesh = plsc.VectorSubcoreMesh(**_MESH)

    @functools.partial(
        pl.kernel,
        out_type=jax.ShapeDtypeStruct((2, NP, D), F32),
        mesh=mesh,
        compiler_params=pltpu.CompilerParams(needs_layout_passes=False),
        scratch_types=[
            pltpu.VMEM((TPW, CHUNK), I32),
            pltpu.VMEM((8, CHUNK), I32),
            pltpu.VMEM((8, CHUNK), I32),
            pltpu.VMEM((2, CHUNK, D), F32),
            pltpu.VMEM((VPT, D), F32),
            pltpu.VMEM_SHARED((ACCB, D), F32),
            pltpu.SemaphoreType.DMA,
            pltpu.SemaphoreType.DMA,
        ],
    )
    def kb(pidx_hbm, xep_hbm, out_hbm,
           pix, idxg, idxs, gbuf, stage, acc, gsem, ssem):
        c = lax.axis_index("c")
        s = lax.axis_index("s")
        base = (c * 16 + s) * TPW
        pltpu.sync_copy(pidx_hbm.at[pl.ds(base, TPW)], pix)

        def zstage(i, carry):
            for j in range(D // 16):
                stage[i, pl.ds(j * 16, 16)] = jnp.zeros((16,), F32)
            return carry

        def fire_s(t):
            pltpu.async_copy(gbuf.at[t & 1], acc.at[idxs.at[t & 7]], ssem,
                             add=True)

        def wait_s(t):
            pltpu.make_async_copy(
                gbuf.at[t & 1], acc.at[idxs.at[t & 7]], ssem).wait()

        def fire_g(t):
            pltpu.async_copy(xep_hbm.at[idxg.at[t & 7]], gbuf.at[t & 1],
                             gsem)

        def wait_g(t):
            pltpu.make_async_copy(
                xep_hbm.at[idxg.at[t & 7]], gbuf.at[t & 1], gsem).wait()

        for p in range(NSUB):
            def prep(t, p=p):
                w = t & 7
                for j in range(CHUNK // 16):
                    pk = pix[t, pl.ds(j * 16, 16)]
                    e16 = pk & 8191
                    n16 = (pk >> 13) - p * VH
                    ok = jnp.logical_and(n16 >= 0, n16 < VH)
                    idxg[w, pl.ds(j * 16, 16)] = jnp.minimum(e16, E - 1)
                    idxs[w, pl.ds(j * 16, 16)] = jnp.where(ok, n16, VH)

            lax.fori_loop(0, VPT, zstage, 0)
            pltpu.sync_copy(stage, acc.at[pl.ds(s * 328, 320)])
            pltpu.sync_copy(stage.at[pl.ds(0, 8)],
                            acc.at[pl.ds(s * 328 + 320, 8)])
            plsc.subcore_barrier()

            prep(0)
            fire_g(0)

            def body(t, carry, prep=prep):
                wait_g(t)
                fire_s(t)

                @pl.when(t >= 1)
                def _():
                    wait_s(t - 1)

                @pl.when(t + 1 < TPW)
                def _():
                    prep(t + 1)
                    fire_g(t + 1)

                return carry

            lax.fori_loop(0, TPW, body, 0)
            wait_s(TPW - 1)
            plsc.subcore_barrier()

            pltpu.sync_copy(acc.at[pl.ds(s * VPT, VPT)], stage)
            pltpu.sync_copy(stage,
                            out_hbm.at[c, pl.ds(p * VH + s * VPT, VPT)])
            plsc.subcore_barrier()

    return kb(pidx, xep)


def _ca_body(p_ref, c0_ref, c1_ref, dege_ref, xep_ref):
    sums = p_ref[0] + p_ref[1]
    cnt = c0_ref[...] + c1_ref[...]
    xep_ref[...] = sums * (dege_ref[...] / jnp.maximum(cnt, 1.0))


def _combine_a(pa, c0, c1, dege):
    return pl.pallas_call(
        _ca_body,
        out_shape=jax.ShapeDtypeStruct((E, D), F32),
    )(pa, c0, c1, dege)


def _cb_body(pb_ref, degv_ref, out_ref):
    out_ref[...] = (pb_ref[0, 0:N, :] + pb_ref[1, 0:N, :]) * degv_ref[...]


def _combine_b(pb, degv):
    return pl.pallas_call(
        _cb_body,
        out_shape=jax.ShapeDtypeStruct((N, D), F32),
    )(pb, degv)


def kernel(X, vertex, edges, H, W):
    pad = R * CHUNK - NNZ
    vertex = vertex.astype(I32)
    edges = edges.astype(I32)
    packed = vertex * 8192 + edges
    packed = jnp.reshape(
        jnp.concatenate([packed, jnp.full((pad,), N * 8192 + E, I32)]),
        (R, CHUNK))

    xw = _kernel_mm(X, W)
    degv, dege_row = _kernel_deg(H)
    dege = jnp.reshape(dege_row, (E, 1))

    pa, cnt = _sc_phase_a(packed, xw)
    cnt = jnp.reshape(cnt, (2, E, 1))
    xep = _combine_a(pa, cnt[0], cnt[1], dege)
    pb = _sc_phase_b(packed, xep)
    return _combine_b(pb, degv)


# speedup vs baseline: 1.0906x; 1.0906x over previous
import functools

import jax
import jax.numpy as jnp
from jax import lax
from jax.experimental import pallas as pl
from jax.experimental.pallas import tpu as pltpu
from jax.experimental.pallas import tpu_sc as plsc

N = 10000
E = 4096
NNZ = 320000
D = 128

CHUNK = 128
TPW = 80
R = 32 * TPW

EPT = E // 16
CROWS = E // D
VH = 5120
NSUB = 2
NP = NSUB * VH
ACC = 4112
ACCB = 5248
VPT = VH // 16

F32 = jnp.float32
I32 = jnp.int32

_MESH = dict(core_axis_name="c", subcore_axis_name="s")


_B1 = 400
_G1 = N // _B1


def _k1_body(x_ref, w_ref, h_ref, xw_ref, degv_ref, dege_ref):
    i = pl.program_id(0)
    xw_ref[...] = jnp.dot(x_ref[...], w_ref[...], preferred_element_type=F32)

    h = h_ref[...]
    rv = jnp.sum(h, axis=1, keepdims=True)
    r = lax.rsqrt(rv)
    degv_ref[...] = jnp.where(jnp.isinf(r), 1.0, r)

    ce = jnp.sum(h, axis=0, keepdims=True)

    @pl.when(i == 0)
    def _():
        dege_ref[...] = jnp.zeros_like(dege_ref)

    dege_ref[...] += ce

    @pl.when(i == _G1 - 1)
    def _():
        dege_ref[...] = lax.rsqrt(dege_ref[...])


def _kernel1(X, W, H):
    return pl.pallas_call(
        _k1_body,
        grid=(_G1,),
        in_specs=[
            pl.BlockSpec((_B1, D), lambda i: (i, 0)),
            pl.BlockSpec((D, D), lambda i: (0, 0)),
            pl.BlockSpec((_B1, E), lambda i: (i, 0)),
        ],
        out_specs=[
            pl.BlockSpec((_B1, D), lambda i: (i, 0)),
            pl.BlockSpec((_B1, 1), lambda i: (i, 0)),
            pl.BlockSpec((1, E), lambda i: (0, 0)),
        ],
        out_shape=[
            jax.ShapeDtypeStruct((N, D), F32),
            jax.ShapeDtypeStruct((N, 1), F32),
            jax.ShapeDtypeStruct((1, E), F32),
        ],
    )(X, W, H)


def _sc_phase_a(pidx, table):
    mesh = plsc.VectorSubcoreMesh(**_MESH)

    @functools.partial(
        pl.kernel,
        out_type=(
            jax.ShapeDtypeStruct((2, E, D), F32),
            jax.ShapeDtypeStruct((2, CROWS, D), F32),
        ),
        mesh=mesh,
        compiler_params=pltpu.CompilerParams(needs_layout_passes=False),
        scratch_types=[
            pltpu.VMEM((TPW, CHUNK), I32),
            pltpu.VMEM((8, CHUNK), I32),
            pltpu.VMEM((8, CHUNK), I32),
            pltpu.VMEM((2, CHUNK, D), F32),
            pltpu.VMEM((EPT, D), F32),
            pltpu.VMEM((CROWS, D), F32),
            pltpu.VMEM_SHARED((ACC, D), F32),
            pltpu.SemaphoreType.DMA,
            pltpu.SemaphoreType.DMA,
        ],
    )
    def ka(pidx_hbm, table_hbm, sums_hbm, cnt_hbm,
           pix, idxg, idxs, gbuf, stage, hist, acc, gsem, ssem):
        c = lax.axis_index("c")
        s = lax.axis_index("s")
        base = (c * 16 + s) * TPW
        pltpu.sync_copy(pidx_hbm.at[pl.ds(base, TPW)], pix)

        def zstage(i, carry):
            for j in range(D // 16):
                stage[i, pl.ds(j * 16, 16)] = jnp.zeros((16,), F32)
            return carry

        lax.fori_loop(0, EPT, zstage, 0)

        @pl.when(s < 15)
        def _():
            pltpu.sync_copy(stage, acc.at[pl.ds(s * 264, 256)])
            pltpu.sync_copy(stage.at[pl.ds(0, 8)],
                            acc.at[pl.ds(s * 264 + 256, 8)])

        @pl.when(s == 15)
        def _():
            pltpu.sync_copy(stage.at[pl.ds(0, 152)], acc.at[pl.ds(3960, 152)])

        def zhist(i, carry):
            for j in range(D // 16):
                hist[i, pl.ds(j * 16, 16)] = jnp.zeros((16,), F32)
            return carry

        lax.fori_loop(0, CROWS, zhist, 0)
        plsc.subcore_barrier()

        def prep(t):
            w = t & 7
            for j in range(CHUNK // 16):
                pk = pix[t, pl.ds(j * 16, 16)]
                idxs[w, pl.ds(j * 16, 16)] = pk & 8191
                idxg[w, pl.ds(j * 16, 16)] = jnp.minimum(pk >> 13, N - 1)

        def fire_g(t):
            pltpu.async_copy(table_hbm.at[idxg.at[t & 7]], gbuf.at[t & 1],
                             gsem)

        def wait_g(t):
            pltpu.make_async_copy(
                table_hbm.at[idxg.at[t & 7]], gbuf.at[t & 1], gsem).wait()

        def fire_s(t):
            pltpu.async_copy(gbuf.at[t & 1], acc.at[idxs.at[t & 7]], ssem,
                             add=True)

        def wait_s(t):
            pltpu.make_async_copy(
                gbuf.at[t & 1], acc.at[idxs.at[t & 7]], ssem).wait()

        prep(0)
        fire_g(0)

        def body(t, carry):
            wait_g(t)
            fire_s(t)

            @pl.when(t >= 1)
            def _():
                wait_s(t - 1)

            @pl.when(t + 1 < TPW)
            def _():
                prep(t + 1)
                fire_g(t + 1)

            for j in range(CHUNK // 16):
                idx16 = pix[t, pl.ds(j * 16, 16)] & 8191
                plsc.addupdate_scatter(
                    hist, [idx16 >> 7, idx16 & 127], jnp.ones((16,), F32),
                    mask=idx16 < E)
            return carry

        lax.fori_loop(0, TPW, body, 0)
        wait_s(TPW - 1)
        plsc.subcore_barrier()

        pltpu.sync_copy(hist, sums_hbm.at[c, pl.ds(s * CROWS, CROWS)])
        plsc.subcore_barrier()

        @pl.when(s < 4)
        def _():
            for r in range(8):
                for j in range(D // 16):
                    hist[r, pl.ds(j * 16, 16)] = jnp.zeros((16,), F32)
            for src in range(16):
                pltpu.sync_copy(
                    sums_hbm.at[c, pl.ds(src * CROWS + 8 * s, 8)],
                    hist.at[pl.ds(8, 8)])
                for r in range(8):
                    for j in range(D // 16):
                        hist[r, pl.ds(j * 16, 16)] = (
                            hist[r, pl.ds(j * 16, 16)]
                            + hist[8 + r, pl.ds(j * 16, 16)])
            pltpu.sync_copy(hist.at[pl.ds(0, 8)],
                            cnt_hbm.at[c, pl.ds(8 * s, 8)])

        plsc.subcore_barrier()

        pltpu.sync_copy(acc.at[pl.ds(s * EPT, EPT)], stage)
        pltpu.sync_copy(stage, sums_hbm.at[c, pl.ds(s * EPT, EPT)])

    return ka(pidx, table)


def _sc_phase_b(pidx, xep):
    mesh = plsc.VectorSubcoreMesh(**_MESH)

    @functools.partial(
        pl.kernel,
        out_type=jax.ShapeDtypeStruct((2, NP, D), F32),
        mesh=mesh,
        compiler_params=pltpu.CompilerParams(needs_layout_passes=False),
        scratch_types=[
            pltpu.VMEM((TPW, CHUNK), I32),
            pltpu.VMEM((8, CHUNK), I32),
            pltpu.VMEM((8, CHUNK), I32),
            pltpu.VMEM((2, CHUNK, D), F32),
            pltpu.VMEM((VPT, D), F32),
            pltpu.VMEM_SHARED((ACCB, D), F32),
            pltpu.SemaphoreType.DMA,
            pltpu.SemaphoreType.DMA,
        ],
    )
    def kb(pidx_hbm, xep_hbm, out_hbm,
           pix, idxg, idxs, gbuf, stage, acc, gsem, ssem):
        c = lax.axis_index("c")
        s = lax.axis_index("s")
        base = (c * 16 + s) * TPW
        pltpu.sync_copy(pidx_hbm.at[pl.ds(base, TPW)], pix)

        def zstage(i, carry):
            for j in range(D // 16):
                stage[i, pl.ds(j * 16, 16)] = jnp.zeros((16,), F32)
            return carry

        def fire_s(t):
            pltpu.async_copy(gbuf.at[t & 1], acc.at[idxs.at[t & 7]], ssem,
                             add=True)

        def wait_s(t):
            pltpu.make_async_copy(
                gbuf.at[t & 1], acc.at[idxs.at[t & 7]], ssem).wait()

        def fire_g(t):
            pltpu.async_copy(xep_hbm.at[idxg.at[t & 7]], gbuf.at[t & 1],
                             gsem)

        def wait_g(t):
            pltpu.make_async_copy(
                xep_hbm.at[idxg.at[t & 7]], gbuf.at[t & 1], gsem).wait()

        for p in range(NSUB):
            def prep(t, p=p):
                w = t & 7
                for j in range(CHUNK // 16):
                    pk = pix[t, pl.ds(j * 16, 16)]
                    e16 = pk & 8191
                    n16 = (pk >> 13) - p * VH
                    ok = jnp.logical_and(n16 >= 0, n16 < VH)
                    idxg[w, pl.ds(j * 16, 16)] = jnp.minimum(e16, E - 1)
                    idxs[w, pl.ds(j * 16, 16)] = jnp.where(ok, n16, VH)

            lax.fori_loop(0, VPT, zstage, 0)
            pltpu.sync_copy(stage, acc.at[pl.ds(s * 328, 320)])
            pltpu.sync_copy(stage.at[pl.ds(0, 8)],
                            acc.at[pl.ds(s * 328 + 320, 8)])
            plsc.subcore_barrier()

            prep(0)
            fire_g(0)

            def body(t, carry, prep=prep):
                wait_g(t)
                fire_s(t)

                @pl.when(t >= 1)
                def _():
                    wait_s(t - 1)

                @pl.when(t + 1 < TPW)
                def _():
                    prep(t + 1)
                    fire_g(t + 1)

                return carry

            lax.fori_loop(0, TPW, body, 0)
            wait_s(TPW - 1)
            plsc.subcore_barrier()

            pltpu.sync_copy(acc.at[pl.ds(s * VPT, VPT)], stage)
            pltpu.sync_copy(stage,
                            out_hbm.at[c, pl.ds(p * VH + s * VPT, VPT)])
            plsc.subcore_barrier()

    return kb(pidx, xep)


def _ca_body(p_ref, c0_ref, c1_ref, dege_ref, xep_ref):
    sums = p_ref[0] + p_ref[1]
    cnt = c0_ref[...] + c1_ref[...]
    xep_ref[...] = sums * (dege_ref[...] / jnp.maximum(cnt, 1.0))


def _combine_a(pa, c0, c1, dege):
    return pl.pallas_call(
        _ca_body,
        out_shape=jax.ShapeDtypeStruct((E, D), F32),
    )(pa, c0, c1, dege)


def _cb_body(pb_ref, degv_ref, out_ref):
    out_ref[...] = (pb_ref[0, 0:N, :] + pb_ref[1, 0:N, :]) * degv_ref[...]


def _combine_b(pb, degv):
    return pl.pallas_call(
        _cb_body,
        out_shape=jax.ShapeDtypeStruct((N, D), F32),
    )(pb, degv)


def kernel(X, vertex, edges, H, W):
    pad = R * CHUNK - NNZ
    vertex = vertex.astype(I32)
    edges = edges.astype(I32)
    packed = vertex * 8192 + edges
    packed = jnp.reshape(
        jnp.concatenate([packed, jnp.full((pad,), N * 8192 + E, I32)]),
        (R, CHUNK))

    xw, degv, dege_row = _kernel1(X, W, H)
    dege = jnp.reshape(dege_row, (E, 1))

    pa, cnt = _sc_phase_a(packed, xw)
    cnt = jnp.reshape(cnt, (2, E, 1))
    xep = _combine_a(pa, cnt[0], cnt[1], dege)
    pb = _sc_phase_b(packed, xep)
    return _combine_b(pb, degv)
